# Initial kernel scaffold; baseline (speedup 1.0000x reference)
import jax
import jax.numpy as jnp
from jax.experimental import pallas as pl


def kernel(x, edge_index, edge_attr, Wr1, br1, Wr2, br2, Wo1, bo1, Wo2, bo2):
    raise NotImplementedError("write your pallas kernel here")


# speedup vs baseline: 2.7772x; 2.7772x over previous
import functools

import jax
import jax.numpy as jnp
from jax import lax
from jax.experimental import pallas as pl
from jax.experimental.pallas import tpu as pltpu
from jax.experimental.pallas import tpu_sc as plsc

_NC = 2
_NS = 16
_NW = _NC * _NS

_PAD = 48


def _proj_body(x_ref, w_ref, b_ref, pd_ref, ps_ref):
    out = jnp.dot(x_ref[...], w_ref[...], preferred_element_type=jnp.float32)
    out = out + b_ref[...]
    pd_ref[...] = out[:, :_PAD]
    ps_ref[...] = out[:, _PAD:]


def _proj(x, wcat, bcat, bn):
    n = x.shape[0]
    nd = x.shape[1]
    grid = n // bn
    return pl.pallas_call(
        _proj_body,
        grid=(grid,),
        in_specs=[
            pl.BlockSpec((bn, nd), lambda i: (i, 0)),
            pl.BlockSpec((nd, 2 * _PAD), lambda i: (0, 0)),
            pl.BlockSpec((1, 2 * _PAD), lambda i: (0, 0)),
        ],
        out_specs=[
            pl.BlockSpec((bn, _PAD), lambda i: (i, 0)),
            pl.BlockSpec((bn, _PAD), lambda i: (i, 0)),
        ],
        out_shape=[
            jax.ShapeDtypeStruct((n, _PAD), jnp.float32),
            jax.ShapeDtypeStruct((n, _PAD), jnp.float32),
        ],
    )(x, wcat, bcat)


def _gather(pd, ps, dst, src, chunk):
    e_num = dst.shape[0]
    per_w = e_num // _NW
    n_ch = per_w // chunk
    mesh = plsc.VectorSubcoreMesh(
        core_axis_name="c", subcore_axis_name="s",
        num_cores=_NC, num_subcores=_NS)

    @functools.partial(
        pl.kernel,
        out_type=(
            jax.ShapeDtypeStruct((e_num, _PAD), jnp.float32),
            jax.ShapeDtypeStruct((e_num, _PAD), jnp.float32),
        ),
        mesh=mesh,
        compiler_params=pltpu.CompilerParams(use_tc_tiling_on_sc=False),
        scratch_types=[
            pltpu.VMEM((chunk,), jnp.int32),
            pltpu.VMEM((chunk,), jnp.int32),
            pltpu.VMEM((chunk, _PAD), jnp.float32),
            pltpu.VMEM((chunk, _PAD), jnp.float32),
            pltpu.SemaphoreType.DMA,
            pltpu.SemaphoreType.DMA,
        ],
    )
    def k(pd_hbm, ps_hbm, dst_hbm, src_hbm, gd_hbm, gs_hbm,
          idxd_v, idxs_v, rowd_v, rows_v, semd, sems):
        wid = lax.axis_index("s") * _NC + lax.axis_index("c")

        def chunk_body(ch, carry):
            base = wid * per_w + ch * chunk
            pltpu.sync_copy(dst_hbm.at[pl.ds(base, chunk)], idxd_v)
            pltpu.sync_copy(src_hbm.at[pl.ds(base, chunk)], idxs_v)
            cpd = pltpu.async_copy(pd_hbm.at[idxd_v], rowd_v, semd)
            cps = pltpu.async_copy(ps_hbm.at[idxs_v], rows_v, sems)
            cpd.wait()
            pltpu.sync_copy(rowd_v, gd_hbm.at[pl.ds(base, chunk)])
            cps.wait()
            pltpu.sync_copy(rows_v, gs_hbm.at[pl.ds(base, chunk)])
            return carry

        lax.fori_loop(0, n_ch, chunk_body, 0)

    return k(pd, ps, dst, src)


def _edge_body(gd_ref, gs_ref, e_ref, we_ref, wr2_ref, br2_ref, m_ref):
    h = gd_ref[...] + gs_ref[...]
    h = h + jnp.dot(e_ref[...], we_ref[...], preferred_element_type=jnp.float32)
    h = jax.nn.relu(h)
    m_ref[...] = (
        jnp.dot(h, wr2_ref[...], preferred_element_type=jnp.float32)
        + br2_ref[...])


def _edge_mlp(gd, gs, e, we_p, wr2_p, br2, be):
    e_num, ed = e.shape
    grid = e_num // be
    return pl.pallas_call(
        _edge_body,
        grid=(grid,),
        in_specs=[
            pl.BlockSpec((be, _PAD), lambda i: (i, 0)),
            pl.BlockSpec((be, _PAD), lambda i: (i, 0)),
            pl.BlockSpec((be, ed), lambda i: (i, 0)),
            pl.BlockSpec((ed, _PAD), lambda i: (0, 0)),
            pl.BlockSpec((_PAD, ed), lambda i: (0, 0)),
            pl.BlockSpec((1, ed), lambda i: (0, 0)),
        ],
        out_specs=pl.BlockSpec((be, ed), lambda i: (i, 0)),
        out_shape=jax.ShapeDtypeStruct((e_num, ed), jnp.float32),
    )(gd, gs, e, we_p, wr2_p, br2)


def _scatter(m, dst, zeros_rows, n_nodes, chunk):
    e_num, ed = m.shape
    per_w = e_num // _NW
    n_ch = per_w // chunk
    zc = zeros_rows.shape[0]
    nz = n_nodes // zc
    mesh = plsc.VectorSubcoreMesh(
        core_axis_name="c", subcore_axis_name="s",
        num_cores=_NC, num_subcores=_NS)

    @functools.partial(
        pl.kernel,
        out_type=jax.ShapeDtypeStruct((_NC, n_nodes, ed), jnp.float32),
        mesh=mesh,
        compiler_params=pltpu.CompilerParams(use_tc_tiling_on_sc=False),
        scratch_types=[
            pltpu.VMEM((chunk,), jnp.int32),
            pltpu.VMEM((chunk, ed), jnp.float32),
            pltpu.VMEM_SHARED((n_nodes, ed), jnp.float32),
        ],
    )
    def k(m_hbm, dst_hbm, z_hbm, agg_hbm, idx_v, rows_v, acc_s):
        cid = lax.axis_index("c")
        sid = lax.axis_index("s")
        wid = sid * _NC + cid

        @pl.when(sid < nz)
        def _():
            pltpu.sync_copy(z_hbm, acc_s.at[pl.ds(sid * zc, zc)])

        plsc.subcore_barrier()

        def chunk_body(ch, carry):
            base = wid * per_w + ch * chunk
            pltpu.sync_copy(dst_hbm.at[pl.ds(base, chunk)], idx_v)
            pltpu.sync_copy(m_hbm.at[pl.ds(base, chunk)], rows_v)
            pltpu.sync_copy(rows_v, acc_s.at[idx_v], add=True)
            return carry

        lax.fori_loop(0, n_ch, chunk_body, 0)

        plsc.subcore_barrier()

        @pl.when(sid < nz)
        def _():
            pltpu.sync_copy(acc_s.at[pl.ds(sid * zc, zc)],
                            agg_hbm.at[cid, pl.ds(sid * zc, zc)])

    return k(m, dst, zeros_rows)


def _node_body(x_ref, a0_ref, a1_ref, wox_ref, woa_ref, bo1_ref,
               wo2_ref, bo2_ref, out_ref):
    x = x_ref[...]
    agg = a0_ref[...] + a1_ref[...]
    nh = jnp.dot(x, wox_ref[...], preferred_element_type=jnp.float32)
    nh = nh + jnp.dot(agg, woa_ref[...], preferred_element_type=jnp.float32)
    nh = jax.nn.relu(nh + bo1_ref[...])
    dx = jnp.dot(nh, wo2_ref[...], preferred_element_type=jnp.float32)
    dx = dx + bo2_ref[...]
    out_ref[...] = 0.5 * x + 0.5 * jax.nn.relu(dx)


def _node_update(x, a0, a1, wox, woa, bo1, wo2, bo2, bn):
    n, nd = x.shape
    ed = a0.shape[1]
    oh = wox.shape[1]
    grid = n // bn
    return pl.pallas_call(
        _node_body,
        grid=(grid,),
        in_specs=[
            pl.BlockSpec((bn, nd), lambda i: (i, 0)),
            pl.BlockSpec((bn, ed), lambda i: (i, 0)),
            pl.BlockSpec((bn, ed), lambda i: (i, 0)),
            pl.BlockSpec((nd, oh), lambda i: (0, 0)),
            pl.BlockSpec((ed, oh), lambda i: (0, 0)),
            pl.BlockSpec((1, oh), lambda i: (0, 0)),
            pl.BlockSpec((oh, nd), lambda i: (0, 0)),
            pl.BlockSpec((1, nd), lambda i: (0, 0)),
        ],
        out_specs=pl.BlockSpec((bn, nd), lambda i: (i, 0)),
        out_shape=jax.ShapeDtypeStruct((n, nd), jnp.float32),
    )(x, a0, a1, wox, woa, bo1, wo2, bo2)


def kernel(x, edge_index, edge_attr, Wr1, br1, Wr2, br2, Wo1, bo1, Wo2, bo2):
    n, nd = x.shape
    e_num, ed = edge_attr.shape
    num_layers = Wr1.shape[0]
    rh = Wr1.shape[2]
    oh = Wo1.shape[2]
    alpha = 0.5

    src = edge_index[0]
    dst = edge_index[1]

    bn = 1000
    be = 8000
    sc_chunk = 1000
    zc = 1000

    padc = _PAD - rh
    zeros_rows = jnp.zeros((zc, ed), jnp.float32)

    e = edge_attr
    for l in range(num_layers):
        wd = Wr1[l, :nd]
        ws = Wr1[l, nd:2 * nd]
        we = Wr1[l, 2 * nd:]
        wcat = jnp.concatenate([
            jnp.pad(wd, ((0, 0), (0, padc))),
            jnp.pad(ws, ((0, 0), (0, padc))),
        ], axis=1)
        bcat = jnp.concatenate([
            jnp.pad(br1[l], (0, padc)), jnp.zeros((_PAD,), jnp.float32)
        ]).reshape(1, 2 * _PAD)
        we_p = jnp.pad(we, ((0, 0), (0, padc)))
        wr2_p = jnp.pad(Wr2[l], ((0, padc), (0, 0)))

        pd, ps = _proj(x, wcat, bcat, bn)
        gd, gs = _gather(pd, ps, dst, src, sc_chunk)
        m = _edge_mlp(gd, gs, e, we_p, wr2_p, br2[l].reshape(1, ed), be)
        agg2 = _scatter(m, dst, zeros_rows, n, sc_chunk)
        x = _node_update(x, agg2[0], agg2[1],
                         Wo1[l, :nd], Wo1[l, nd:], bo1[l].reshape(1, oh),
                         Wo2[l], bo2[l].reshape(1, nd), bn)
        e = m
    del alpha
    return x, e
